# baseline (speedup 1.0000x reference)
import functools

import jax
import jax.numpy as jnp
from jax import lax
from jax.experimental import pallas as pl
from jax.experimental.pallas import tpu as pltpu
from jax.experimental.pallas import tpu_sc as plsc

N = 10000
E = 160000
B = 128
EPS = 1e-5

NC, NS = 2, 16
NW = NC * NS
NBN = 320
N_PAD = NW * NBN

CH = 32000
NCHUNK = E // CH
E_CAP = E + NCHUNK * 256

GB = 128
EW = E // NW
NBW = (EW + GB - 1) // GB
DSL = 3

_mesh = plsc.VectorSubcoreMesh(core_axis_name="c", subcore_axis_name="s",
                               num_cores=NC, num_subcores=NS)


def _wid():
    return lax.axis_index("s") * NC + lax.axis_index("c")


def _build_lists_body(dst_hbm, lists_hbm, counts_hbm, dbuf, idbuf, cbuf):
    w = _wid()
    base = w * NBN
    iota = lax.iota(jnp.int32, 16)
    total = jnp.int32(0)
    for c in range(NCHUNK):
        pltpu.sync_copy(dst_hbm.at[pl.ds(c * CH, CH)], dbuf)

        def grp(j, cnt):
            d = dbuf[pl.ds(j * 16, 16)]
            m = (d >= base) & (d < base + NBN)
            eid = (c * CH) + j * 16 + iota
            packed = lax.shift_left(eid, 9) | ((d - base) & 511)
            mi = m.astype(jnp.int32)
            inc = plsc.cumsum(mi)
            pos = cnt + inc - mi
            plsc.store_scatter(idbuf, [pos], packed, mask=m)
            return cnt + jnp.sum(mi)

        cnt = lax.fori_loop(0, CH // 16, grp, jnp.int32(0))
        dummy = jnp.full((16,), (c * CH) << 9 | NBN, jnp.int32)
        for k in range(16):
            idbuf[pl.ds(cnt + k * 16, 16)] = dummy
        padded = ((cnt + 255) // 256) * 256

        def flush(bi, _):
            pltpu.sync_copy(idbuf.at[pl.ds(bi * 128, 128)],
                            lists_hbm.at[w, pl.ds(total + bi * 128, 128)])
            return jnp.int32(0)

        lax.fori_loop(0, padded // 128, flush, jnp.int32(0))
        total = total + padded
    for k in range(8):
        cbuf[pl.ds(k * 16, 16)] = jnp.where((iota == 0) & (k == 0), total, 0)
    pltpu.sync_copy(cbuf, counts_hbm.at[w])


_build_lists = pl.kernel(
    _build_lists_body,
    out_type=[jax.ShapeDtypeStruct((NW, E_CAP), jnp.int32),
              jax.ShapeDtypeStruct((NW, 128), jnp.int32)],
    mesh=_mesh,
    compiler_params=pltpu.CompilerParams(needs_layout_passes=False),
    scratch_types=[pltpu.VMEM((CH,), jnp.int32),
                   pltpu.VMEM((CH + 256,), jnp.int32),
                   pltpu.VMEM((128,), jnp.int32)],
)


def _gather_body(Wd, p_hbm, q_hbm, dst_hbm, src_hbm, op_hbm, oq_hbm, *sc):
    dbuf, sbuf = sc[0], sc[1]
    pbs, qbs = sc[2:2 + DSL], sc[2 + DSL:2 + 2 * DSL]
    s0 = 2 + 2 * DSL
    sgp, sgq = sc[s0:s0 + DSL], sc[s0 + DSL:s0 + 2 * DSL]
    swp, swq = sc[s0 + 2 * DSL:s0 + 3 * DSL], sc[s0 + 3 * DSL:s0 + 4 * DSL]

    w = _wid()
    e0 = w * EW
    pltpu.sync_copy(dst_hbm.at[pl.ds(e0, EW)], dbuf)
    pltpu.sync_copy(src_hbm.at[pl.ds(e0, EW)], sbuf)

    offs = [min(b * GB, EW - GB) for b in range(NBW)]

    def issue(b, k):
        o = offs[b]
        pltpu.async_copy(p_hbm.at[dbuf.at[pl.ds(o, GB)]], pbs[k], sgp[k])
        pltpu.async_copy(q_hbm.at[sbuf.at[pl.ds(o, GB)]], qbs[k], sgq[k])

    def wait_gather(b, k):
        o = offs[b]
        pltpu.make_async_copy(p_hbm.at[dbuf.at[pl.ds(o, GB)]], pbs[k],
                              sgp[k]).wait()
        pltpu.make_async_copy(q_hbm.at[sbuf.at[pl.ds(o, GB)]], qbs[k],
                              sgq[k]).wait()

    def issue_write(b, k):
        o = offs[b]
        pltpu.async_copy(pbs[k], op_hbm.at[pl.ds(e0 + o, GB)], swp[k])
        pltpu.async_copy(qbs[k], oq_hbm.at[pl.ds(e0 + o, GB)], swq[k])

    def wait_write(b, k):
        o = offs[b]
        pltpu.make_async_copy(pbs[k], op_hbm.at[pl.ds(e0 + o, GB)],
                              swp[k]).wait()
        pltpu.make_async_copy(qbs[k], oq_hbm.at[pl.ds(e0 + o, GB)],
                              swq[k]).wait()

    for b in range(min(DSL, NBW)):
        issue(b, b % DSL)
    for b in range(NBW):
        k = b % DSL
        wait_gather(b, k)
        issue_write(b, k)
        if b + DSL < NBW:
            wait_write(b, k)
            issue(b + DSL, k)
    for b in range(max(NBW - DSL, 0), NBW):
        wait_write(b, b % DSL)


def _make_gather(Wd, dt):
    return pl.kernel(
        functools.partial(_gather_body, Wd),
        out_type=[jax.ShapeDtypeStruct((E, Wd), dt),
                  jax.ShapeDtypeStruct((E, Wd), dt)],
        mesh=_mesh,
        compiler_params=pltpu.CompilerParams(needs_layout_passes=False),
        scratch_types=[pltpu.VMEM((EW,), jnp.int32),
                       pltpu.VMEM((EW,), jnp.int32)]
        + [pltpu.VMEM((GB, Wd), dt) for _ in range(2 * DSL)]
        + [pltpu.SemaphoreType.DMA for _ in range(4 * DSL)],
    )


def _segmax_body(F, SB, h_hbm, lists_hbm, counts_hbm, c_hbm, acc, pkbuf,
                 ids0, ids1, ldv0, ldv1, hb0, hb1, cbuf, sem0, sem1):
    w = _wid()
    iota = lax.iota(jnp.int32, 16)
    zf = jnp.zeros((16,), jnp.float32)

    def zrow(r, _):
        for f in range(F // 16):
            acc[r, pl.ds(f * 16, 16)] = zf
        return jnp.int32(0)

    lax.fori_loop(0, NBN + 1, zrow, jnp.int32(0))

    pltpu.sync_copy(counts_hbm.at[w], cbuf)
    total = jnp.sum(jnp.where(iota == 0, cbuf[pl.ds(0, 16)], 0))
    nb = total // SB

    def stage(b, ids, ldv, hb, sem):
        pltpu.sync_copy(lists_hbm.at[w, pl.ds(b * SB, SB)], pkbuf)

        def unp(k, _):
            v = pkbuf[pl.ds(k * 16, 16)]
            ids[pl.ds(k * 16, 16)] = lax.shift_right_logical(v, 9)
            ldv[pl.ds(k * 16, 16)] = v & 511
            return jnp.int32(0)

        lax.fori_loop(0, SB // 16, unp, jnp.int32(0))
        pltpu.async_copy(h_hbm.at[ids], hb, sem)

    def process(ids, ldv, hb, sem):
        pltpu.make_async_copy(h_hbm.at[ids], hb, sem).wait()

        def grp(g, _):
            ov = ldv[pl.ds(g * 16, 16)]
            for k in range(16):
                ld = lax.squeeze(lax.slice(ov, (k,), (k + 1,)), (0,))
                j = g * 16 + k
                for f in range(F // 16):
                    a = acc[ld, pl.ds(f * 16, 16)]
                    h = hb[j, pl.ds(f * 16, 16)]
                    acc[ld, pl.ds(f * 16, 16)] = jnp.maximum(a, h)
            return jnp.int32(0)

        lax.fori_loop(0, SB // 16, grp, jnp.int32(0))

    @pl.when(nb > 0)
    def _():
        stage(0, ids0, ldv0, hb0, sem0)

    def pair(i, _):
        b = i * 2

        @pl.when(b + 1 < nb)
        def _():
            stage(b + 1, ids1, ldv1, hb1, sem1)

        process(ids0, ldv0, hb0, sem0)

        @pl.when(b + 2 < nb)
        def _():
            stage(b + 2, ids0, ldv0, hb0, sem0)

        @pl.when(b + 1 < nb)
        def _():
            process(ids1, ldv1, hb1, sem1)

        return jnp.int32(0)

    lax.fori_loop(0, (nb + 1) // 2, pair, jnp.int32(0))
    pltpu.sync_copy(acc.at[pl.ds(0, NBN)], c_hbm.at[pl.ds(w * NBN, NBN)])


def _make_segmax(F, SB):
    return pl.kernel(
        functools.partial(_segmax_body, F, SB),
        out_type=jax.ShapeDtypeStruct((N_PAD, F), jnp.float32),
        mesh=_mesh,
        compiler_params=pltpu.CompilerParams(needs_layout_passes=False),
        scratch_types=[pltpu.VMEM((NBN + 1, F), jnp.float32),
                       pltpu.VMEM((SB,), jnp.int32),
                       pltpu.VMEM((SB,), jnp.int32),
                       pltpu.VMEM((SB,), jnp.int32),
                       pltpu.VMEM((SB,), jnp.int32),
                       pltpu.VMEM((SB,), jnp.int32),
                       pltpu.VMEM((SB, F), jnp.float32),
                       pltpu.VMEM((SB, F), jnp.float32),
                       pltpu.VMEM((128,), jnp.int32),
                       pltpu.SemaphoreType.DMA,
                       pltpu.SemaphoreType.DMA],
    )


def _b16(x):
    i = lax.bitcast_convert_type(x, jnp.int32)
    r = i + 0x7FFF + (lax.shift_right_logical(i, 16) & 1)
    return lax.shift_right_logical(r, 16)


def _pack_pair(x, H):
    return _b16(x[:, :H]) | lax.shift_left(_b16(x[:, H:]), 16)


def _unpack_lo(w32):
    return lax.bitcast_convert_type(lax.shift_left(w32, 16), jnp.float32)


def _unpack_hi(w32):
    return lax.bitcast_convert_type(w32 & jnp.int32(-65536), jnp.float32)


def _gn_proj_body(x_ref, bt_ref, ones_ref, gw_ref, gb_ref, gms_ref,
                  wa_ref, wb_ref, b1_ref, p_ref, q_ref):
    x = x_ref[...]
    bt = bt_ref[...]
    oh = (bt == lax.broadcasted_iota(jnp.int32, (N_PAD, B), 1)).astype(jnp.float32)
    dn = (((0,), (0,)), ((), ()))
    cnt = jnp.maximum(lax.dot_general(oh, ones_ref[...], dn)[:, 0:1], 1.0)
    mean = lax.dot_general(oh, x, dn) / cnt
    sub = x - gms_ref[...] * jnp.dot(oh, mean, preferred_element_type=jnp.float32)
    var = lax.dot_general(oh, sub * sub, dn) / cnt
    varn = jnp.dot(oh, var, preferred_element_type=jnp.float32)
    xn = gw_ref[...] * sub * lax.rsqrt(varn + EPS) + gb_ref[...]
    p_ref[...] = jnp.dot(xn, wa_ref[...], preferred_element_type=jnp.float32) + b1_ref[...]
    q_ref[...] = jnp.dot(xn, wb_ref[...], preferred_element_type=jnp.float32)


def _gn_proj(x_pad, bt, ones8, gw, gb, gms, wa, wb, b1):
    fo = wa.shape[1]
    return pl.pallas_call(
        _gn_proj_body,
        out_shape=[jax.ShapeDtypeStruct((N_PAD, fo), jnp.float32),
                   jax.ShapeDtypeStruct((N_PAD, fo), jnp.float32)],
    )(x_pad, bt, ones8, gw, gb, gms, wa, wb, b1)


def _proj_body(pack, c_ref, wa_ref, wb_ref, b1_ref, p_ref, q_ref):
    c = c_ref[...]
    p = jnp.dot(c, wa_ref[...], preferred_element_type=jnp.float32) + b1_ref[...]
    q = jnp.dot(c, wb_ref[...], preferred_element_type=jnp.float32)
    if pack:
        fo = p.shape[1]
        p_ref[...] = _pack_pair(p, fo // 2)
        q_ref[...] = _pack_pair(q, fo // 2)
    else:
        p_ref[...] = p
        q_ref[...] = q


def _proj(c, wa, wb, b1, pack=False):
    fi, fo = wa.shape
    ho = fo // 2 if pack else fo
    dt = jnp.int32 if pack else jnp.float32
    bn = 1024
    grid = N_PAD // bn
    return pl.pallas_call(
        functools.partial(_proj_body, pack),
        grid=(grid,),
        in_specs=[pl.BlockSpec((bn, fi), lambda i: (i, 0)),
                  pl.BlockSpec((fi, fo), lambda i: (0, 0)),
                  pl.BlockSpec((fi, fo), lambda i: (0, 0)),
                  pl.BlockSpec((1, fo), lambda i: (0, 0))],
        out_specs=[pl.BlockSpec((bn, ho), lambda i: (i, 0)),
                   pl.BlockSpec((bn, ho), lambda i: (i, 0))],
        out_shape=[jax.ShapeDtypeStruct((N_PAD, ho), dt),
                   jax.ShapeDtypeStruct((N_PAD, ho), dt)],
    )(c, wa, wb, b1)


def _mlp_f32_body(gp_ref, gq_ref, w2_ref, b2_ref, w3_ref, b3_ref, o_ref):
    h = jnp.maximum(gp_ref[...] + gq_ref[...], 0.0).astype(jnp.bfloat16)
    h = jnp.maximum(jnp.dot(h, w2_ref[...], preferred_element_type=jnp.float32)
                    + b2_ref[...], 0.0).astype(jnp.bfloat16)
    h = jnp.maximum(jnp.dot(h, w3_ref[...], preferred_element_type=jnp.float32)
                    + b3_ref[...], 0.0)
    o_ref[...] = h


def _edge_mlp_f32(gp, gq, w2, b2, w3, b3):
    f = w2.shape[0]
    fm = w2.shape[1]
    fo = w3.shape[1]
    be = 1000
    grid = E // be
    return pl.pallas_call(
        _mlp_f32_body,
        grid=(grid,),
        in_specs=[pl.BlockSpec((be, f), lambda i: (i, 0)),
                  pl.BlockSpec((be, f), lambda i: (i, 0)),
                  pl.BlockSpec((f, fm), lambda i: (0, 0)),
                  pl.BlockSpec((1, fm), lambda i: (0, 0)),
                  pl.BlockSpec((fm, fo), lambda i: (0, 0)),
                  pl.BlockSpec((1, fo), lambda i: (0, 0))],
        out_specs=pl.BlockSpec((be, fo), lambda i: (i, 0)),
        out_shape=jax.ShapeDtypeStruct((E, fo), jnp.float32),
    )(gp, gq, w2, b2, w3, b3)


def _mlp_packed_body(op_ref, oq_ref, w2a_ref, w2b_ref, b2_ref, w3_ref, b3_ref,
                     o_ref):
    wp = op_ref[...]
    wq = oq_ref[...]
    lo = jnp.maximum(_unpack_lo(wp) + _unpack_lo(wq), 0.0).astype(jnp.bfloat16)
    hi = jnp.maximum(_unpack_hi(wp) + _unpack_hi(wq), 0.0).astype(jnp.bfloat16)
    h = (jnp.dot(lo, w2a_ref[...], preferred_element_type=jnp.float32)
         + jnp.dot(hi, w2b_ref[...], preferred_element_type=jnp.float32)
         + b2_ref[...])
    h = jnp.maximum(h, 0.0).astype(jnp.bfloat16)
    h = jnp.maximum(jnp.dot(h, w3_ref[...], preferred_element_type=jnp.float32)
                    + b3_ref[...], 0.0)
    o_ref[...] = h


def _edge_mlp_packed(op, oq, w2a, w2b, b2, w3, b3):
    hh = w2a.shape[0]
    fm = w2a.shape[1]
    fo = w3.shape[1]
    be = 1000
    grid = E // be
    return pl.pallas_call(
        _mlp_packed_body,
        grid=(grid,),
        in_specs=[pl.BlockSpec((be, hh), lambda i: (i, 0)),
                  pl.BlockSpec((be, hh), lambda i: (i, 0)),
                  pl.BlockSpec((hh, fm), lambda i: (0, 0)),
                  pl.BlockSpec((hh, fm), lambda i: (0, 0)),
                  pl.BlockSpec((1, fm), lambda i: (0, 0)),
                  pl.BlockSpec((fm, fo), lambda i: (0, 0)),
                  pl.BlockSpec((1, fo), lambda i: (0, 0))],
        out_specs=pl.BlockSpec((be, fo), lambda i: (i, 0)),
        out_shape=jax.ShapeDtypeStruct((E, fo), jnp.float32),
    )(op, oq, w2a, w2b, b2, w3, b3)


def _head_body(c1_ref, c2_ref, c3_ref, bt_ref, ones_ref, gi_ref,
               w1a_ref, w1b_ref, w1c_ref, w1g_ref, bd1_ref,
               w2_ref, bd2_ref, wo_ref, bo_ref, o_ref):
    bt = bt_ref[...]
    oh = (bt == lax.broadcasted_iota(jnp.int32, (N_PAD, B), 1)).astype(jnp.float32)
    dn = (((0,), (0,)), ((), ()))
    cnt = jnp.maximum(lax.dot_general(oh, ones_ref[...], dn)[:, 0:1], 1.0)
    p1 = lax.dot_general(oh, c1_ref[...], dn) / cnt
    p2 = lax.dot_general(oh, c2_ref[...], dn) / cnt
    p3 = lax.dot_general(oh, c3_ref[...], dn) / cnt
    d1 = (jnp.dot(p1, w1a_ref[...], preferred_element_type=jnp.float32)
          + jnp.dot(p2, w1b_ref[...], preferred_element_type=jnp.float32)
          + jnp.dot(p3, w1c_ref[...], preferred_element_type=jnp.float32)
          + jnp.dot(gi_ref[...], w1g_ref[...], preferred_element_type=jnp.float32)
          + bd1_ref[...])
    h1 = jnp.where(d1 > 0, d1, 0.01 * d1)
    d2 = jnp.dot(h1, w2_ref[...], preferred_element_type=jnp.float32) + bd2_ref[...]
    h2 = jnp.where(d2 > 0, d2, 0.01 * d2)
    o_ref[...] = jnp.dot(h2, wo_ref[...], preferred_element_type=jnp.float32) + bo_ref[...]


def _head(c1, c2, c3, bt, ones8, gi, w1a, w1b, w1c, w1g, bd1, w2, bd2, wo, bo):
    return pl.pallas_call(
        _head_body,
        out_shape=jax.ShapeDtypeStruct((B, 128), jnp.float32),
    )(c1, c2, c3, bt, ones8, gi, w1a, w1b, w1c, w1g, bd1, w2, bd2, wo, bo)


_gather128f = _make_gather(128, jnp.float32)
_gather128i = _make_gather(128, jnp.int32)
_segmax128 = _make_segmax(128, 256)
_segmax256 = _make_segmax(256, 64)


def kernel(x, edge_index, graph_input, batch, gn_weight, gn_bias, gn_mean_scale,
           W11, b11, W12, b12, W13, b13, W21, b21, W22, b22, W23, b23,
           W31, b31, W32, b32, W33, b33, bn0_g, bn0_b, Wd1, bd1, bn1_g, bn1_b,
           Wd2, bd2, bn2_g, bn2_b, Wo, bo):
    f32 = jnp.float32
    bf16 = jnp.bfloat16
    src = edge_index[0]
    dst = edge_index[1]

    x_pad = jnp.pad(x, ((0, N_PAD - N), (0, 7)))
    bt = jnp.pad(batch, (0, N_PAD - N), constant_values=-1).reshape(N_PAD, 1)
    ones8 = jnp.ones((N_PAD, 8), f32)

    def padrc(w, r, c):
        return jnp.pad(w, ((0, r - w.shape[0]), (0, c - w.shape[1])))

    wa1 = padrc(W11[:9] - W11[9:], 16, 128)
    wb1 = padrc(W11[9:], 16, 128)
    gw = jnp.pad(gn_weight, (0, 7)).reshape(1, 16)
    gb = jnp.pad(gn_bias, (0, 7)).reshape(1, 16)
    gms = jnp.pad(gn_mean_scale, (0, 7)).reshape(1, 16)

    lists, counts = _build_lists(dst)

    b11p = jnp.pad(b11, (0, 64)).reshape(1, 128)
    p, q = _gn_proj(x_pad, bt, ones8, gw, gb, gms, wa1, wb1, b11p)
    gp, gq = _gather128f(p, q, dst, src)
    h = _edge_mlp_f32(gp, gq, padrc(W12, 128, 64).astype(bf16),
                      b12.reshape(1, 64), padrc(W13, 64, 128).astype(bf16),
                      jnp.pad(b13, (0, 64)).reshape(1, 128))
    c1 = _segmax128(h, lists, counts)

    p, q = _proj(c1, padrc(W21[:64] - W21[64:], 128, 128),
                 padrc(W21[64:], 128, 128), b21.reshape(1, 128))
    gp, gq = _gather128f(p, q, dst, src)
    h = _edge_mlp_f32(gp, gq, W22.astype(bf16), b22.reshape(1, 128),
                      W23.astype(bf16), b23.reshape(1, 128))
    c2 = _segmax128(h, lists, counts)

    p, q = _proj(c2, W31[:128] - W31[128:], W31[128:], b31.reshape(1, 256),
                 pack=True)
    op, oq = _gather128i(p, q, dst, src)
    h = _edge_mlp_packed(op, oq, W32[:128].astype(bf16), W32[128:].astype(bf16),
                         b32.reshape(1, 256), W33.astype(bf16),
                         b33.reshape(1, 256))
    c3 = _segmax256(h, lists, counts)

    s0 = (bn0_g / jnp.sqrt(1.0 + EPS))[:, None] * Wd1
    bd1f = (bd1 + bn0_b @ Wd1).reshape(1, 256)
    w1a = jnp.pad(s0[0:64], ((0, 64), (0, 0)))
    w1b, w1c = s0[64:192], s0[192:448]
    w1g = jnp.pad(s0[448:452], ((0, 60), (0, 0)))
    gi = jnp.pad(graph_input, ((0, 0), (0, 60)))
    s1 = (bn1_g / jnp.sqrt(1.0 + EPS))[:, None] * Wd2
    bd2f = (bd2 + bn1_b @ Wd2).reshape(1, 128)
    s2 = (bn2_g / jnp.sqrt(1.0 + EPS))[:, None] * Wo
    bof = (bo + bn2_b @ Wo).reshape(1, 4)
    wo_pad = jnp.pad(s2, ((0, 0), (0, 124)))
    bo_pad = jnp.pad(bof, ((0, 0), (0, 124)))

    out = _head(c1, c2, c3, bt, ones8, gi, w1a, w1b, w1c, w1g, bd1f,
                s1, bd2f, wo_pad, bo_pad)
    return out[:, :4]


# speedup vs baseline: 1.1840x; 1.1840x over previous
import functools

import jax
import jax.numpy as jnp
from jax import lax
from jax.experimental import pallas as pl
from jax.experimental.pallas import tpu as pltpu
from jax.experimental.pallas import tpu_sc as plsc

N = 10000
E = 160000
B = 128
EPS = 1e-5

NC, NS = 2, 16
NW = NC * NS
NBN = 320
N_PAD = NW * NBN

CH = 32000
NCHUNK = E // CH
E_CAP = E + NCHUNK * 128

GB = 128
EW = E // NW
NBW = (EW + GB - 1) // GB
DSL = 3

_mesh = plsc.VectorSubcoreMesh(core_axis_name="c", subcore_axis_name="s",
                               num_cores=NC, num_subcores=NS)


def _wid():
    return lax.axis_index("s") * NC + lax.axis_index("c")


def _build_lists_body(dst_hbm, lists_hbm, counts_hbm, dbuf, idbuf, cbuf):
    w = _wid()
    base = w * NBN
    iota = lax.iota(jnp.int32, 16)
    total = jnp.int32(0)
    for c in range(NCHUNK):
        pltpu.sync_copy(dst_hbm.at[pl.ds(c * CH, CH)], dbuf)

        def grp(j, cnt):
            d = dbuf[pl.ds(j * 16, 16)]
            m = (d >= base) & (d < base + NBN)
            eid = (c * CH) + j * 16 + iota
            packed = lax.shift_left(eid, 9) | ((d - base) & 511)
            mi = m.astype(jnp.int32)
            inc = plsc.cumsum(mi)
            pos = cnt + inc - mi
            plsc.store_scatter(idbuf, [pos], packed, mask=m)
            return cnt + jnp.sum(mi)

        cnt = lax.fori_loop(0, CH // 16, grp, jnp.int32(0))
        dummy = jnp.full((16,), (c * CH) << 9 | NBN, jnp.int32)
        for k in range(8):
            idbuf[pl.ds(cnt + k * 16, 16)] = dummy
        padded = ((cnt + 127) // 128) * 128

        def flush(bi, _):
            pltpu.sync_copy(idbuf.at[pl.ds(bi * 128, 128)],
                            lists_hbm.at[w, pl.ds(total + bi * 128, 128)])
            return jnp.int32(0)

        lax.fori_loop(0, padded // 128, flush, jnp.int32(0))
        total = total + padded
    for k in range(8):
        cbuf[pl.ds(k * 16, 16)] = jnp.where((iota == 0) & (k == 0), total, 0)
    pltpu.sync_copy(cbuf, counts_hbm.at[w])


_build_lists = pl.kernel(
    _build_lists_body,
    out_type=[jax.ShapeDtypeStruct((NW, E_CAP), jnp.int32),
              jax.ShapeDtypeStruct((NW, 128), jnp.int32)],
    mesh=_mesh,
    compiler_params=pltpu.CompilerParams(needs_layout_passes=False),
    scratch_types=[pltpu.VMEM((CH,), jnp.int32),
                   pltpu.VMEM((CH + 128,), jnp.int32),
                   pltpu.VMEM((128,), jnp.int32)],
)


def _gather_body(Wd, p_hbm, q_hbm, dst_hbm, src_hbm, op_hbm, oq_hbm, *sc):
    dbuf, sbuf = sc[0], sc[1]
    pbs, qbs = sc[2:2 + DSL], sc[2 + DSL:2 + 2 * DSL]
    s0 = 2 + 2 * DSL
    sgp, sgq = sc[s0:s0 + DSL], sc[s0 + DSL:s0 + 2 * DSL]
    swp, swq = sc[s0 + 2 * DSL:s0 + 3 * DSL], sc[s0 + 3 * DSL:s0 + 4 * DSL]

    w = _wid()
    e0 = w * EW
    pltpu.sync_copy(dst_hbm.at[pl.ds(e0, EW)], dbuf)
    pltpu.sync_copy(src_hbm.at[pl.ds(e0, EW)], sbuf)

    offs = [min(b * GB, EW - GB) for b in range(NBW)]

    def issue(b, k):
        o = offs[b]
        pltpu.async_copy(p_hbm.at[dbuf.at[pl.ds(o, GB)]], pbs[k], sgp[k])
        pltpu.async_copy(q_hbm.at[sbuf.at[pl.ds(o, GB)]], qbs[k], sgq[k])

    def wait_gather(b, k):
        o = offs[b]
        pltpu.make_async_copy(p_hbm.at[dbuf.at[pl.ds(o, GB)]], pbs[k],
                              sgp[k]).wait()
        pltpu.make_async_copy(q_hbm.at[sbuf.at[pl.ds(o, GB)]], qbs[k],
                              sgq[k]).wait()

    def issue_write(b, k):
        o = offs[b]
        pltpu.async_copy(pbs[k], op_hbm.at[pl.ds(e0 + o, GB)], swp[k])
        pltpu.async_copy(qbs[k], oq_hbm.at[pl.ds(e0 + o, GB)], swq[k])

    def wait_write(b, k):
        o = offs[b]
        pltpu.make_async_copy(pbs[k], op_hbm.at[pl.ds(e0 + o, GB)],
                              swp[k]).wait()
        pltpu.make_async_copy(qbs[k], oq_hbm.at[pl.ds(e0 + o, GB)],
                              swq[k]).wait()

    for b in range(min(DSL, NBW)):
        issue(b, b % DSL)
    for b in range(NBW):
        k = b % DSL
        wait_gather(b, k)
        issue_write(b, k)
        if b + DSL < NBW:
            wait_write(b, k)
            issue(b + DSL, k)
    for b in range(max(NBW - DSL, 0), NBW):
        wait_write(b, b % DSL)


def _make_gather(Wd, dt):
    return pl.kernel(
        functools.partial(_gather_body, Wd),
        out_type=[jax.ShapeDtypeStruct((E, Wd), dt),
                  jax.ShapeDtypeStruct((E, Wd), dt)],
        mesh=_mesh,
        compiler_params=pltpu.CompilerParams(needs_layout_passes=False),
        scratch_types=[pltpu.VMEM((EW,), jnp.int32),
                       pltpu.VMEM((EW,), jnp.int32)]
        + [pltpu.VMEM((GB, Wd), dt) for _ in range(2 * DSL)]
        + [pltpu.SemaphoreType.DMA for _ in range(4 * DSL)],
    )


def _segmax_body(F, SB, h_hbm, lists_hbm, counts_hbm, c_hbm, acc, pkbuf,
                 ids0, ids1, ldv0, ldv1, hb0, hb1, cbuf, sem0, sem1):
    w = _wid()
    iota = lax.iota(jnp.int32, 16)
    zf = jnp.zeros((16,), jnp.float32)

    def zrow(r, _):
        for f in range(F // 16):
            acc[r, pl.ds(f * 16, 16)] = zf
        return jnp.int32(0)

    lax.fori_loop(0, NBN + 1, zrow, jnp.int32(0))

    pltpu.sync_copy(counts_hbm.at[w], cbuf)
    total = jnp.sum(jnp.where(iota == 0, cbuf[pl.ds(0, 16)], 0))
    nb = total // SB

    def stage(b, ids, ldv, hb, sem):
        pltpu.sync_copy(lists_hbm.at[w, pl.ds(b * SB, SB)], pkbuf)

        def unp(k, _):
            v = pkbuf[pl.ds(k * 16, 16)]
            ids[pl.ds(k * 16, 16)] = lax.shift_right_logical(v, 9)
            ldv[pl.ds(k * 16, 16)] = v & 511
            return jnp.int32(0)

        lax.fori_loop(0, SB // 16, unp, jnp.int32(0))
        pltpu.async_copy(h_hbm.at[ids], hb, sem)

    def process(ids, ldv, hb, sem):
        pltpu.make_async_copy(h_hbm.at[ids], hb, sem).wait()

        def grp(g, _):
            ov = ldv[pl.ds(g * 16, 16)]
            for k in range(16):
                ld = lax.squeeze(lax.slice(ov, (k,), (k + 1,)), (0,))
                j = g * 16 + k
                for f in range(F // 16):
                    a = acc[ld, pl.ds(f * 16, 16)]
                    h = hb[j, pl.ds(f * 16, 16)]
                    acc[ld, pl.ds(f * 16, 16)] = jnp.maximum(a, h)
            return jnp.int32(0)

        lax.fori_loop(0, SB // 16, grp, jnp.int32(0))

    @pl.when(nb > 0)
    def _():
        stage(0, ids0, ldv0, hb0, sem0)

    def pair(i, _):
        b = i * 2

        @pl.when(b + 1 < nb)
        def _():
            stage(b + 1, ids1, ldv1, hb1, sem1)

        process(ids0, ldv0, hb0, sem0)

        @pl.when(b + 2 < nb)
        def _():
            stage(b + 2, ids0, ldv0, hb0, sem0)

        @pl.when(b + 1 < nb)
        def _():
            process(ids1, ldv1, hb1, sem1)

        return jnp.int32(0)

    lax.fori_loop(0, (nb + 1) // 2, pair, jnp.int32(0))
    pltpu.sync_copy(acc.at[pl.ds(0, NBN)], c_hbm.at[pl.ds(w * NBN, NBN)])


def _make_segmax(F, SB):
    return pl.kernel(
        functools.partial(_segmax_body, F, SB),
        out_type=jax.ShapeDtypeStruct((N_PAD, F), jnp.float32),
        mesh=_mesh,
        compiler_params=pltpu.CompilerParams(needs_layout_passes=False),
        scratch_types=[pltpu.VMEM((NBN + 1, F), jnp.float32),
                       pltpu.VMEM((SB,), jnp.int32),
                       pltpu.VMEM((SB,), jnp.int32),
                       pltpu.VMEM((SB,), jnp.int32),
                       pltpu.VMEM((SB,), jnp.int32),
                       pltpu.VMEM((SB,), jnp.int32),
                       pltpu.VMEM((SB, F), jnp.float32),
                       pltpu.VMEM((SB, F), jnp.float32),
                       pltpu.VMEM((128,), jnp.int32),
                       pltpu.SemaphoreType.DMA,
                       pltpu.SemaphoreType.DMA],
    )


def _b16(x):
    i = lax.bitcast_convert_type(x, jnp.int32)
    r = i + 0x7FFF + (lax.shift_right_logical(i, 16) & 1)
    return lax.shift_right_logical(r, 16)


def _pack_pair(x, H):
    return _b16(x[:, :H]) | lax.shift_left(_b16(x[:, H:]), 16)


def _unpack_lo(w32):
    return lax.bitcast_convert_type(lax.shift_left(w32, 16), jnp.float32)


def _unpack_hi(w32):
    return lax.bitcast_convert_type(w32 & jnp.int32(-65536), jnp.float32)


def _gn_proj_body(x_ref, bt_ref, ones_ref, gw_ref, gb_ref, gms_ref,
                  wa_ref, wb_ref, b1_ref, p_ref, q_ref):
    x = x_ref[...]
    bt = bt_ref[...]
    oh = (bt == lax.broadcasted_iota(jnp.int32, (N_PAD, B), 1)).astype(jnp.float32)
    dn = (((0,), (0,)), ((), ()))
    cnt = jnp.maximum(lax.dot_general(oh, ones_ref[...], dn)[:, 0:1], 1.0)
    mean = lax.dot_general(oh, x, dn) / cnt
    sub = x - gms_ref[...] * jnp.dot(oh, mean, preferred_element_type=jnp.float32)
    var = lax.dot_general(oh, sub * sub, dn) / cnt
    varn = jnp.dot(oh, var, preferred_element_type=jnp.float32)
    xn = gw_ref[...] * sub * lax.rsqrt(varn + EPS) + gb_ref[...]
    p_ref[...] = jnp.dot(xn, wa_ref[...], preferred_element_type=jnp.float32) + b1_ref[...]
    q_ref[...] = jnp.dot(xn, wb_ref[...], preferred_element_type=jnp.float32)


def _gn_proj(x_pad, bt, ones8, gw, gb, gms, wa, wb, b1):
    fo = wa.shape[1]
    return pl.pallas_call(
        _gn_proj_body,
        out_shape=[jax.ShapeDtypeStruct((N_PAD, fo), jnp.float32),
                   jax.ShapeDtypeStruct((N_PAD, fo), jnp.float32)],
    )(x_pad, bt, ones8, gw, gb, gms, wa, wb, b1)


def _proj_body(pack, c_ref, wa_ref, wb_ref, b1_ref, p_ref, q_ref):
    c = c_ref[...]
    p = jnp.dot(c, wa_ref[...], preferred_element_type=jnp.float32) + b1_ref[...]
    q = jnp.dot(c, wb_ref[...], preferred_element_type=jnp.float32)
    if pack:
        fo = p.shape[1]
        p_ref[...] = _pack_pair(p, fo // 2)
        q_ref[...] = _pack_pair(q, fo // 2)
    else:
        p_ref[...] = p
        q_ref[...] = q


def _proj(c, wa, wb, b1, pack=False):
    fi, fo = wa.shape
    ho = fo // 2 if pack else fo
    dt = jnp.int32 if pack else jnp.float32
    bn = 1024
    grid = N_PAD // bn
    return pl.pallas_call(
        functools.partial(_proj_body, pack),
        grid=(grid,),
        in_specs=[pl.BlockSpec((bn, fi), lambda i: (i, 0)),
                  pl.BlockSpec((fi, fo), lambda i: (0, 0)),
                  pl.BlockSpec((fi, fo), lambda i: (0, 0)),
                  pl.BlockSpec((1, fo), lambda i: (0, 0))],
        out_specs=[pl.BlockSpec((bn, ho), lambda i: (i, 0)),
                   pl.BlockSpec((bn, ho), lambda i: (i, 0))],
        out_shape=[jax.ShapeDtypeStruct((N_PAD, ho), dt),
                   jax.ShapeDtypeStruct((N_PAD, ho), dt)],
    )(c, wa, wb, b1)


def _mlp_f32_body(gp_ref, gq_ref, w2_ref, b2_ref, w3_ref, b3_ref, o_ref):
    h = jnp.maximum(gp_ref[...] + gq_ref[...], 0.0).astype(jnp.bfloat16)
    h = jnp.maximum(jnp.dot(h, w2_ref[...], preferred_element_type=jnp.float32)
                    + b2_ref[...], 0.0).astype(jnp.bfloat16)
    h = jnp.maximum(jnp.dot(h, w3_ref[...], preferred_element_type=jnp.float32)
                    + b3_ref[...], 0.0)
    o_ref[...] = h


def _edge_mlp_f32(gp, gq, w2, b2, w3, b3):
    f = w2.shape[0]
    fm = w2.shape[1]
    fo = w3.shape[1]
    be = 1000
    grid = E // be
    return pl.pallas_call(
        _mlp_f32_body,
        grid=(grid,),
        in_specs=[pl.BlockSpec((be, f), lambda i: (i, 0)),
                  pl.BlockSpec((be, f), lambda i: (i, 0)),
                  pl.BlockSpec((f, fm), lambda i: (0, 0)),
                  pl.BlockSpec((1, fm), lambda i: (0, 0)),
                  pl.BlockSpec((fm, fo), lambda i: (0, 0)),
                  pl.BlockSpec((1, fo), lambda i: (0, 0))],
        out_specs=pl.BlockSpec((be, fo), lambda i: (i, 0)),
        out_shape=jax.ShapeDtypeStruct((E, fo), jnp.float32),
    )(gp, gq, w2, b2, w3, b3)


def _mlp_packed_body(op_ref, oq_ref, w2a_ref, w2b_ref, b2_ref, w3_ref, b3_ref,
                     o_ref):
    wp = op_ref[...]
    wq = oq_ref[...]
    lo = jnp.maximum(_unpack_lo(wp) + _unpack_lo(wq), 0.0).astype(jnp.bfloat16)
    hi = jnp.maximum(_unpack_hi(wp) + _unpack_hi(wq), 0.0).astype(jnp.bfloat16)
    h = (jnp.dot(lo, w2a_ref[...], preferred_element_type=jnp.float32)
         + jnp.dot(hi, w2b_ref[...], preferred_element_type=jnp.float32)
         + b2_ref[...])
    h = jnp.maximum(h, 0.0).astype(jnp.bfloat16)
    h = jnp.maximum(jnp.dot(h, w3_ref[...], preferred_element_type=jnp.float32)
                    + b3_ref[...], 0.0)
    o_ref[...] = h


def _edge_mlp_packed(op, oq, w2a, w2b, b2, w3, b3):
    hh = w2a.shape[0]
    fm = w2a.shape[1]
    fo = w3.shape[1]
    be = 1000
    grid = E // be
    return pl.pallas_call(
        _mlp_packed_body,
        grid=(grid,),
        in_specs=[pl.BlockSpec((be, hh), lambda i: (i, 0)),
                  pl.BlockSpec((be, hh), lambda i: (i, 0)),
                  pl.BlockSpec((hh, fm), lambda i: (0, 0)),
                  pl.BlockSpec((hh, fm), lambda i: (0, 0)),
                  pl.BlockSpec((1, fm), lambda i: (0, 0)),
                  pl.BlockSpec((fm, fo), lambda i: (0, 0)),
                  pl.BlockSpec((1, fo), lambda i: (0, 0))],
        out_specs=pl.BlockSpec((be, fo), lambda i: (i, 0)),
        out_shape=jax.ShapeDtypeStruct((E, fo), jnp.float32),
    )(op, oq, w2a, w2b, b2, w3, b3)


def _head_body(c1_ref, c2_ref, c3_ref, bt_ref, ones_ref, gi_ref,
               w1a_ref, w1b_ref, w1c_ref, w1g_ref, bd1_ref,
               w2_ref, bd2_ref, wo_ref, bo_ref, o_ref):
    bt = bt_ref[...]
    oh = (bt == lax.broadcasted_iota(jnp.int32, (N_PAD, B), 1)).astype(jnp.float32)
    dn = (((0,), (0,)), ((), ()))
    cnt = jnp.maximum(lax.dot_general(oh, ones_ref[...], dn)[:, 0:1], 1.0)
    p1 = lax.dot_general(oh, c1_ref[...], dn) / cnt
    p2 = lax.dot_general(oh, c2_ref[...], dn) / cnt
    p3 = lax.dot_general(oh, c3_ref[...], dn) / cnt
    d1 = (jnp.dot(p1, w1a_ref[...], preferred_element_type=jnp.float32)
          + jnp.dot(p2, w1b_ref[...], preferred_element_type=jnp.float32)
          + jnp.dot(p3, w1c_ref[...], preferred_element_type=jnp.float32)
          + jnp.dot(gi_ref[...], w1g_ref[...], preferred_element_type=jnp.float32)
          + bd1_ref[...])
    h1 = jnp.where(d1 > 0, d1, 0.01 * d1)
    d2 = jnp.dot(h1, w2_ref[...], preferred_element_type=jnp.float32) + bd2_ref[...]
    h2 = jnp.where(d2 > 0, d2, 0.01 * d2)
    o_ref[...] = jnp.dot(h2, wo_ref[...], preferred_element_type=jnp.float32) + bo_ref[...]


def _head(c1, c2, c3, bt, ones8, gi, w1a, w1b, w1c, w1g, bd1, w2, bd2, wo, bo):
    return pl.pallas_call(
        _head_body,
        out_shape=jax.ShapeDtypeStruct((B, 128), jnp.float32),
    )(c1, c2, c3, bt, ones8, gi, w1a, w1b, w1c, w1g, bd1, w2, bd2, wo, bo)


_gather128f = _make_gather(128, jnp.float32)
_gather128i = _make_gather(128, jnp.int32)
_segmax128 = _make_segmax(128, 128)
_segmax256 = _make_segmax(256, 64)


def kernel(x, edge_index, graph_input, batch, gn_weight, gn_bias, gn_mean_scale,
           W11, b11, W12, b12, W13, b13, W21, b21, W22, b22, W23, b23,
           W31, b31, W32, b32, W33, b33, bn0_g, bn0_b, Wd1, bd1, bn1_g, bn1_b,
           Wd2, bd2, bn2_g, bn2_b, Wo, bo):
    f32 = jnp.float32
    bf16 = jnp.bfloat16
    src = edge_index[0]
    dst = edge_index[1]

    x_pad = jnp.pad(x, ((0, N_PAD - N), (0, 7)))
    bt = jnp.pad(batch, (0, N_PAD - N), constant_values=-1).reshape(N_PAD, 1)
    ones8 = jnp.ones((N_PAD, 8), f32)

    def padrc(w, r, c):
        return jnp.pad(w, ((0, r - w.shape[0]), (0, c - w.shape[1])))

    wa1 = padrc(W11[:9] - W11[9:], 16, 128)
    wb1 = padrc(W11[9:], 16, 128)
    gw = jnp.pad(gn_weight, (0, 7)).reshape(1, 16)
    gb = jnp.pad(gn_bias, (0, 7)).reshape(1, 16)
    gms = jnp.pad(gn_mean_scale, (0, 7)).reshape(1, 16)

    lists, counts = _build_lists(dst)

    b11p = jnp.pad(b11, (0, 64)).reshape(1, 128)
    p, q = _gn_proj(x_pad, bt, ones8, gw, gb, gms, wa1, wb1, b11p)
    gp, gq = _gather128f(p, q, dst, src)
    h = _edge_mlp_f32(gp, gq, padrc(W12, 128, 64).astype(bf16),
                      b12.reshape(1, 64), padrc(W13, 64, 128).astype(bf16),
                      jnp.pad(b13, (0, 64)).reshape(1, 128))
    c1 = _segmax128(h, lists, counts)

    p, q = _proj(c1, padrc(W21[:64] - W21[64:], 128, 128),
                 padrc(W21[64:], 128, 128), b21.reshape(1, 128))
    gp, gq = _gather128f(p, q, dst, src)
    h = _edge_mlp_f32(gp, gq, W22.astype(bf16), b22.reshape(1, 128),
                      W23.astype(bf16), b23.reshape(1, 128))
    c2 = _segmax128(h, lists, counts)

    p, q = _proj(c2, W31[:128] - W31[128:], W31[128:], b31.reshape(1, 256),
                 pack=True)
    op, oq = _gather128i(p, q, dst, src)
    h = _edge_mlp_packed(op, oq, W32[:128].astype(bf16), W32[128:].astype(bf16),
                         b32.reshape(1, 256), W33.astype(bf16),
                         b33.reshape(1, 256))
    c3 = _segmax256(h, lists, counts)

    s0 = (bn0_g / jnp.sqrt(1.0 + EPS))[:, None] * Wd1
    bd1f = (bd1 + bn0_b @ Wd1).reshape(1, 256)
    w1a = jnp.pad(s0[0:64], ((0, 64), (0, 0)))
    w1b, w1c = s0[64:192], s0[192:448]
    w1g = jnp.pad(s0[448:452], ((0, 60), (0, 0)))
    gi = jnp.pad(graph_input, ((0, 0), (0, 60)))
    s1 = (bn1_g / jnp.sqrt(1.0 + EPS))[:, None] * Wd2
    bd2f = (bd2 + bn1_b @ Wd2).reshape(1, 128)
    s2 = (bn2_g / jnp.sqrt(1.0 + EPS))[:, None] * Wo
    bof = (bo + bn2_b @ Wo).reshape(1, 4)
    wo_pad = jnp.pad(s2, ((0, 0), (0, 124)))
    bo_pad = jnp.pad(bof, ((0, 0), (0, 124)))

    out = _head(c1, c2, c3, bt, ones8, gi, w1a, w1b, w1c, w1g, bd1f,
                s1, bd2f, wo_pad, bo_pad)
    return out[:, :4]
